# baseline (speedup 1.0000x reference)
import functools

import jax
import jax.numpy as jnp
from jax import lax
from jax.experimental import pallas as pl
from jax.experimental.pallas import tpu as pltpu
from jax.experimental.pallas import tpu_sc as plsc

OVERSAMP = 4
NV = 64
NPIX_LO = 128
PIXSCALE_LO = 0.1
VEL0 = -500.0
DV = 1000.0 / (NV - 1)
NPIX_HI = NPIX_LO * OVERSAMP
PIXSCALE_HI = PIXSCALE_LO / OVERSAMP
FOV_HALF_HI = 0.5 * (NPIX_HI - 1) * PIXSCALE_HI

NLO = NV * NPIX_LO * NPIX_LO

NC = 2
NS = 16
LANES = 16
NW = NC * NS

M_TOTAL = 262144 * 8
P_PER_W = M_TOTAL // NW
CHUNK = 1024
N_CHUNKS = P_PER_W // CHUNK
N_VECS = CHUNK // LANES
ENTRIES = 8 * CHUNK

_mesh = plsc.VectorSubcoreMesh(
    core_axis_name="c", subcore_axis_name="s", num_cores=NC, num_subcores=NS
)


@functools.partial(
    pl.kernel,
    out_type=jax.ShapeDtypeStruct((NC, NLO), jnp.float32),
    mesh=_mesh,
    scratch_types=[
        pltpu.VMEM((4, CHUNK), jnp.float32),
        pltpu.VMEM((4, CHUNK), jnp.float32),
        pltpu.VMEM((ENTRIES,), jnp.int32),
        pltpu.VMEM((ENTRIES,), jnp.float32),
        pltpu.VMEM((ENTRIES,), jnp.int32),
        pltpu.VMEM((ENTRIES,), jnp.float32),
        pltpu.VMEM_SHARED((NLO,), jnp.float32),
        pltpu.SemaphoreType.DMA,
        pltpu.SemaphoreType.DMA,
        pltpu.SemaphoreType.DMA,
        pltpu.SemaphoreType.DMA,
    ],
)
def _sc_splat(pos_h, vel_h, flx_h, out_h, in_a, in_b,
              idx_a, val_a, idx_b, val_b, cube_sh, sem_a, sem_b,
              sem_ia, sem_ib):
    cid = lax.axis_index("c")
    sid = lax.axis_index("s")
    wid = cid * NS + sid

    inv_p = jnp.float32(1.0 / PIXSCALE_HI)
    off_p = jnp.float32(FOV_HALF_HI / PIXSCALE_HI)
    inv_dv = jnp.float32(1.0 / DV)
    off_v = jnp.float32(-VEL0 / DV)
    sixteenth = jnp.float32(1.0 / 16.0)

    def _zero_body(i, _):
        val_a[pl.ds(i * LANES, LANES)] = jnp.zeros((LANES,), jnp.float32)
        return 0
    lax.fori_loop(0, ENTRIES // LANES, _zero_body, 0)
    words_per_tile = NLO // NS
    zb = 4096
    def _zinit_body(j, _):
        pltpu.sync_copy(
            val_a.at[pl.ds(0, zb)],
            cube_sh.at[pl.ds(sid * words_per_tile + j * zb, zb)],
        )
        return 0
    lax.fori_loop(0, words_per_tile // zb, _zinit_body, 0)
    plsc.subcore_barrier()

    def _chunk_body(ch):
        p0 = wid * P_PER_W + ch * CHUNK
        pltpu.sync_copy(pos_h.at[pl.ds(2 * p0, CHUNK)], ra_v)
        pltpu.sync_copy(pos_h.at[pl.ds(2 * p0 + CHUNK, CHUNK)], de_v)
        pltpu.sync_copy(vel_h.at[pl.ds(p0, CHUNK)], vel_v)
        pltpu.sync_copy(flx_h.at[pl.ds(p0, CHUNK)], flx_v)

        def _vec_body(i, _):
            s = i * LANES
            ra = ra_v[pl.ds(s, LANES)]
            de = de_v[pl.ds(s, LANES)]
            ve = vel_v[pl.ds(s, LANES)]
            fl = flx_v[pl.ds(s, LANES)]

            x = ra * inv_p + off_p
            y = de * inv_p + off_p
            v = ve * inv_dv + off_v

            ix0 = x.astype(jnp.int32)
            iy0 = y.astype(jnp.int32)
            iv0 = v.astype(jnp.int32)
            fx = x - ix0.astype(jnp.float32)
            fy = y - iy0.astype(jnp.float32)
            fv = v - iv0.astype(jnp.float32)

            m = (
                (x >= 0.0) & (x < NPIX_HI - 1)
                & (y >= 0.0) & (y < NPIX_HI - 1)
                & (v >= 0.0) & (v < NV - 1)
            )
            f = jnp.where(m, fl, jnp.float32(0.0)) * sixteenth

            ix0 = jnp.clip(ix0, 0, NPIX_HI - 2)
            iy0 = jnp.clip(iy0, 0, NPIX_HI - 2)
            iv0 = jnp.clip(iv0, 0, NV - 2)

            jx0 = jnp.right_shift(ix0, 2)
            jx1 = jnp.right_shift(ix0 + 1, 2)
            jy0 = jnp.right_shift(iy0, 2)
            jy1 = jnp.right_shift(iy0 + 1, 2)

            av0 = iv0 * (NPIX_LO * NPIX_LO)
            av1 = av0 + (NPIX_LO * NPIX_LO)
            by0 = jy0 * NPIX_LO
            by1 = jy1 * NPIX_LO
            c00 = av0 + by0
            c01 = av0 + by1
            c10 = av1 + by0
            c11 = av1 + by1

            p = f * fv
            h1 = p * fy
            q = f - p
            h0 = q - q * fy
            a1 = h0 * fx
            a0 = h0 - a1
            b1 = h1 * fx
            b0 = h1 - b1

            o = i * (8 * LANES)
            idx_v[pl.ds(o + 0 * LANES, LANES)] = c00 + jx0
            val_v[pl.ds(o + 0 * LANES, LANES)] = a0
            idx_v[pl.ds(o + 1 * LANES, LANES)] = c00 + jx1
            val_v[pl.ds(o + 1 * LANES, LANES)] = a1
            idx_v[pl.ds(o + 2 * LANES, LANES)] = c01 + jx0
            val_v[pl.ds(o + 2 * LANES, LANES)] = a0
            idx_v[pl.ds(o + 3 * LANES, LANES)] = c01 + jx1
            val_v[pl.ds(o + 3 * LANES, LANES)] = a1
            idx_v[pl.ds(o + 4 * LANES, LANES)] = c10 + jx0
            val_v[pl.ds(o + 4 * LANES, LANES)] = b0
            idx_v[pl.ds(o + 5 * LANES, LANES)] = c10 + jx1
            val_v[pl.ds(o + 5 * LANES, LANES)] = b1
            idx_v[pl.ds(o + 6 * LANES, LANES)] = c11 + jx0
            val_v[pl.ds(o + 6 * LANES, LANES)] = b0
            idx_v[pl.ds(o + 7 * LANES, LANES)] = c11 + jx1
            val_v[pl.ds(o + 7 * LANES, LANES)] = b1
            return 0

        lax.fori_loop(0, N_VECS, _vec_body, 0, unroll=2)

    def _scat(idx_v, val_v, sem):
        return pltpu.make_async_copy(val_v, cube_sh.at[idx_v], sem)

    last = jnp.int32(N_CHUNKS - 1)
    for d in _in_descs(jnp.int32(0), in_a, sem_ia):
        d.start()

    def _pair_body(g, _):
        ch_a = g * 2
        ch_b = ch_a + 1
        for d in _in_descs(ch_a, in_a, sem_ia):
            d.wait()
        for d in _in_descs(ch_b, in_b, sem_ib):
            d.start()
        _compute_chunk(in_a, idx_a, val_a)
        @pl.when(g > 0)
        def _():
            _scat(idx_b, val_b, sem_b).wait()
        _scat(idx_a, val_a, sem_a).start(add=True)

        for d in _in_descs(ch_b, in_b, sem_ib):
            d.wait()
        for d in _in_descs(jnp.minimum(ch_b + 1, last), in_a, sem_ia):
            d.start()
        _compute_chunk(in_b, idx_b, val_b)
        _scat(idx_a, val_a, sem_a).wait()
        _scat(idx_b, val_b, sem_b).start(add=True)
        return 0

    lax.fori_loop(0, N_CHUNKS // 2, _pair_body, 0)
    _scat(idx_b, val_b, sem_b).wait()
    for d in _in_descs(last, in_a, sem_ia):
        d.wait()
    plsc.subcore_barrier()

    wb = words_per_tile // 8
    def _wb_body(j, _):
        o = sid * words_per_tile + j * wb
        pltpu.sync_copy(cube_sh.at[pl.ds(o, wb)], out_h.at[cid, pl.ds(o, wb)])
        return 0
    lax.fori_loop(0, 8, _wb_body, 0)


def _merge_body(p_ref, o_ref):
    o_ref[...] = p_ref[0] + p_ref[1]


def kernel(pos_img, vel_chan, flux):
    N, K, _ = pos_img.shape
    assert N * K == M_TOTAL
    m = M_TOTAL
    pos_t = pos_img.transpose(1, 2, 0)
    vel_t = vel_chan.T
    flx_t = flux.T

    partial = _sc_splat(pos_t, vel_t, flx_t)

    merged = pl.pallas_call(
        _merge_body,
        grid=(8,),
        in_specs=[pl.BlockSpec((NC, 8, 16384), lambda i: (0, i, 0))],
        out_specs=pl.BlockSpec((8, 16384), lambda i: (i, 0)),
        out_shape=jax.ShapeDtypeStruct((NV, 16384), jnp.float32),
    )(partial.reshape(NC, NV, 16384))
    return merged.reshape(NV, NPIX_LO, NPIX_LO)


# speedup vs baseline: 1.0037x; 1.0037x over previous
import functools

import jax
import jax.numpy as jnp
from jax import lax
from jax.experimental import pallas as pl
from jax.experimental.pallas import tpu as pltpu
from jax.experimental.pallas import tpu_sc as plsc

OVERSAMP = 4
NV = 64
NPIX_LO = 128
PIXSCALE_LO = 0.1
VEL0 = -500.0
DV = 1000.0 / (NV - 1)
NPIX_HI = NPIX_LO * OVERSAMP
PIXSCALE_HI = PIXSCALE_LO / OVERSAMP
FOV_HALF_HI = 0.5 * (NPIX_HI - 1) * PIXSCALE_HI

NLO = NV * NPIX_LO * NPIX_LO

NC = 2
NS = 16
LANES = 16
NW = NC * NS

M_TOTAL = 262144 * 8
P_PER_W = M_TOTAL // NW
CHUNK = 1024
N_CHUNKS = P_PER_W // CHUNK
N_VECS = CHUNK // LANES
ENTRIES = 8 * CHUNK

_mesh = plsc.VectorSubcoreMesh(
    core_axis_name="c", subcore_axis_name="s", num_cores=NC, num_subcores=NS
)


@functools.partial(
    pl.kernel,
    out_type=jax.ShapeDtypeStruct((NC, NLO), jnp.float32),
    mesh=_mesh,
    scratch_types=[
        pltpu.VMEM((4, CHUNK), jnp.float32),
        pltpu.VMEM((4, CHUNK), jnp.float32),
        pltpu.VMEM((ENTRIES,), jnp.int32),
        pltpu.VMEM((ENTRIES,), jnp.float32),
        pltpu.VMEM((ENTRIES,), jnp.int32),
        pltpu.VMEM((ENTRIES,), jnp.float32),
        pltpu.VMEM_SHARED((NLO,), jnp.float32),
        pltpu.SemaphoreType.DMA,
        pltpu.SemaphoreType.DMA,
        pltpu.SemaphoreType.DMA,
        pltpu.SemaphoreType.DMA,
    ],
)
def _sc_splat(pos_h, vel_h, flx_h, out_h, in_a, in_b,
              idx_a, val_a, idx_b, val_b, cube_sh, sem_a, sem_b,
              sem_ia, sem_ib):
    cid = lax.axis_index("c")
    sid = lax.axis_index("s")
    wid = cid * NS + sid

    inv_p = jnp.float32(1.0 / PIXSCALE_HI)
    off_p = jnp.float32(FOV_HALF_HI / PIXSCALE_HI)
    inv_dv = jnp.float32(1.0 / DV)
    off_v = jnp.float32(-VEL0 / DV)
    sixteenth = jnp.float32(1.0 / 16.0)

    def _zero_body(i, _):
        val_a[pl.ds(i * LANES, LANES)] = jnp.zeros((LANES,), jnp.float32)
        return 0
    lax.fori_loop(0, ENTRIES // LANES, _zero_body, 0)
    words_per_tile = NLO // NS
    zb = 4096
    def _zinit_body(j, _):
        pltpu.sync_copy(
            val_a.at[pl.ds(0, zb)],
            cube_sh.at[pl.ds(sid * words_per_tile + j * zb, zb)],
        )
        return 0
    lax.fori_loop(0, words_per_tile // zb, _zinit_body, 0)
    plsc.subcore_barrier()

    K_PTS = 8
    N_PTS = M_TOTAL // K_PTS
    NPART = N_PTS // NW
    JCH = NPART // CHUNK

    def _in_descs(ch, in_v, sem):
        k = ch // JCH
        j = ch % JCH
        n0 = wid * NPART + j * CHUNK
        return (
            pltpu.make_async_copy(pos_h.at[k, 0, pl.ds(n0, CHUNK)],
                                  in_v.at[0], sem),
            pltpu.make_async_copy(pos_h.at[k, 1, pl.ds(n0, CHUNK)],
                                  in_v.at[1], sem),
            pltpu.make_async_copy(vel_h.at[k, pl.ds(n0, CHUNK)],
                                  in_v.at[2], sem),
            pltpu.make_async_copy(flx_h.at[k, pl.ds(n0, CHUNK)],
                                  in_v.at[3], sem),
        )

    def _compute_chunk(in_v, idx_v, val_v):
        def _vec_body(i, _):
            s = i * LANES
            ra = in_v[0, pl.ds(s, LANES)]
            de = in_v[1, pl.ds(s, LANES)]
            ve = in_v[2, pl.ds(s, LANES)]
            fl = in_v[3, pl.ds(s, LANES)]

            x = ra * inv_p + off_p
            y = de * inv_p + off_p
            v = ve * inv_dv + off_v

            ix0 = x.astype(jnp.int32)
            iy0 = y.astype(jnp.int32)
            iv0 = v.astype(jnp.int32)
            fx = x - ix0.astype(jnp.float32)
            fy = y - iy0.astype(jnp.float32)
            fv = v - iv0.astype(jnp.float32)

            m = (
                (x >= 0.0) & (x < NPIX_HI - 1)
                & (y >= 0.0) & (y < NPIX_HI - 1)
                & (v >= 0.0) & (v < NV - 1)
            )
            f = jnp.where(m, fl, jnp.float32(0.0)) * sixteenth

            ix0 = jnp.clip(ix0, 0, NPIX_HI - 2)
            iy0 = jnp.clip(iy0, 0, NPIX_HI - 2)
            iv0 = jnp.clip(iv0, 0, NV - 2)

            jx0 = jnp.right_shift(ix0, 2)
            jx1 = jnp.right_shift(ix0 + 1, 2)
            jy0 = jnp.right_shift(iy0, 2)
            jy1 = jnp.right_shift(iy0 + 1, 2)

            av0 = iv0 * (NPIX_LO * NPIX_LO)
            av1 = av0 + (NPIX_LO * NPIX_LO)
            by0 = jy0 * NPIX_LO
            by1 = jy1 * NPIX_LO
            c00 = av0 + by0
            c01 = av0 + by1
            c10 = av1 + by0
            c11 = av1 + by1

            p = f * fv
            h1 = p * fy
            q = f - p
            h0 = q - q * fy
            a1 = h0 * fx
            a0 = h0 - a1
            b1 = h1 * fx
            b0 = h1 - b1

            o = i * (8 * LANES)
            idx_v[pl.ds(o + 0 * LANES, LANES)] = c00 + jx0
            val_v[pl.ds(o + 0 * LANES, LANES)] = a0
            idx_v[pl.ds(o + 1 * LANES, LANES)] = c00 + jx1
            val_v[pl.ds(o + 1 * LANES, LANES)] = a1
            idx_v[pl.ds(o + 2 * LANES, LANES)] = c01 + jx0
            val_v[pl.ds(o + 2 * LANES, LANES)] = a0
            idx_v[pl.ds(o + 3 * LANES, LANES)] = c01 + jx1
            val_v[pl.ds(o + 3 * LANES, LANES)] = a1
            idx_v[pl.ds(o + 4 * LANES, LANES)] = c10 + jx0
            val_v[pl.ds(o + 4 * LANES, LANES)] = b0
            idx_v[pl.ds(o + 5 * LANES, LANES)] = c10 + jx1
            val_v[pl.ds(o + 5 * LANES, LANES)] = b1
            idx_v[pl.ds(o + 6 * LANES, LANES)] = c11 + jx0
            val_v[pl.ds(o + 6 * LANES, LANES)] = b0
            idx_v[pl.ds(o + 7 * LANES, LANES)] = c11 + jx1
            val_v[pl.ds(o + 7 * LANES, LANES)] = b1
            return 0

        lax.fori_loop(0, N_VECS, _vec_body, 0)

    def _scat(idx_v, val_v, sem):
        return pltpu.make_async_copy(val_v, cube_sh.at[idx_v], sem)

    last = jnp.int32(N_CHUNKS - 1)
    for d in _in_descs(jnp.int32(0), in_a, sem_ia):
        d.start()

    def _pair_body(g, _):
        ch_a = g * 2
        ch_b = ch_a + 1
        for d in _in_descs(ch_a, in_a, sem_ia):
            d.wait()
        for d in _in_descs(ch_b, in_b, sem_ib):
            d.start()
        _compute_chunk(in_a, idx_a, val_a)
        @pl.when(g > 0)
        def _():
            _scat(idx_b, val_b, sem_b).wait()
        _scat(idx_a, val_a, sem_a).start(add=True)

        for d in _in_descs(ch_b, in_b, sem_ib):
            d.wait()
        for d in _in_descs(jnp.minimum(ch_b + 1, last), in_a, sem_ia):
            d.start()
        _compute_chunk(in_b, idx_b, val_b)
        _scat(idx_a, val_a, sem_a).wait()
        _scat(idx_b, val_b, sem_b).start(add=True)
        return 0

    lax.fori_loop(0, N_CHUNKS // 2, _pair_body, 0)
    _scat(idx_b, val_b, sem_b).wait()
    for d in _in_descs(last, in_a, sem_ia):
        d.wait()
    plsc.subcore_barrier()

    wb = words_per_tile // 8
    def _wb_body(j, _):
        o = sid * words_per_tile + j * wb
        pltpu.sync_copy(cube_sh.at[pl.ds(o, wb)], out_h.at[cid, pl.ds(o, wb)])
        return 0
    lax.fori_loop(0, 8, _wb_body, 0)


def _merge_body(p_ref, o_ref):
    o_ref[...] = p_ref[0] + p_ref[1]


def kernel(pos_img, vel_chan, flux):
    N, K, _ = pos_img.shape
    assert N * K == M_TOTAL
    m = M_TOTAL
    pos_t = pos_img.transpose(1, 2, 0)
    vel_t = vel_chan.T
    flx_t = flux.T

    partial = _sc_splat(pos_t, vel_t, flx_t)

    merged = pl.pallas_call(
        _merge_body,
        grid=(8,),
        in_specs=[pl.BlockSpec((NC, 8, 16384), lambda i: (0, i, 0))],
        out_specs=pl.BlockSpec((8, 16384), lambda i: (i, 0)),
        out_shape=jax.ShapeDtypeStruct((NV, 16384), jnp.float32),
    )(partial.reshape(NC, NV, 16384))
    return merged.reshape(NV, NPIX_LO, NPIX_LO)
